# baseline (speedup 1.0000x reference)
import functools

import jax
import jax.numpy as jnp
from jax import lax
from jax.experimental import pallas as pl
from jax.experimental.pallas import tpu as pltpu
from jax.experimental.pallas import tpu_sc as plsc

N = 16384
L = 50
B_DIM = 16
FEAT = 16

NC = 2
NS = 16
NW = NC * NS

TOTAL = N * L
G = 128
ROWS_PER_W = TOTAL // (NW * G)
CHUNK_ROWS = 8
C = CHUNK_ROWS * G
N_CHUNKS = ROWS_PER_W // CHUNK_ROWS


def _sc_body(idx_hbm, feat_hbm, table_hbm, oe_hbm, oo_hbm, out_hbm,
             idx_v, oe_v, oo_v, rows_v, feat_v, gsem, ssem):
    wid = lax.axis_index("s") * NC + lax.axis_index("c")
    row_base = wid * ROWS_PER_W

    def chunk(i, _):
        row_off = row_base + i * CHUNK_ROWS
        off = row_off * G
        pltpu.sync_copy(idx_hbm.at[pl.ds(row_off, CHUNK_ROWS)], idx_v)
        gathers = []
        for j in range(CHUNK_ROWS):
            gathers.append(
                pltpu.async_copy(
                    table_hbm.at[idx_v.at[j]],
                    rows_v.at[pl.ds(j * G, G)],
                    gsem,
                )
            )
        pltpu.sync_copy(oe_hbm.at[pl.ds(row_off, CHUNK_ROWS)], oe_v)
        pltpu.sync_copy(oo_hbm.at[pl.ds(row_off, CHUNK_ROWS)], oo_v)
        pltpu.sync_copy(feat_hbm.at[pl.ds(off, C)], feat_v)
        scatters = []
        for j in range(CHUNK_ROWS):
            scatters.append(
                pltpu.async_copy(
                    feat_v.at[pl.ds(j * G, G)],
                    out_hbm.at[oo_v.at[j]],
                    ssem,
                )
            )
        for cp in gathers:
            cp.wait()
        for j in range(CHUNK_ROWS):
            scatters.append(
                pltpu.async_copy(
                    rows_v.at[pl.ds(j * G, G)],
                    out_hbm.at[oe_v.at[j]],
                    ssem,
                )
            )
        for cp in scatters:
            cp.wait()
        return ()

    lax.fori_loop(0, N_CHUNKS, chunk, (), unroll=False)


@jax.jit
def _run(idx2d, feat2d, table, oe2d, oo2d):
    kern = pl.kernel(
        _sc_body,
        out_type=jax.ShapeDtypeStruct((2 * TOTAL, B_DIM), jnp.float32),
        mesh=plsc.VectorSubcoreMesh(
            core_axis_name="c", subcore_axis_name="s",
            num_cores=NC, num_subcores=NS,
        ),
        scratch_types=[
            pltpu.VMEM((CHUNK_ROWS, G), jnp.int32),
            pltpu.VMEM((CHUNK_ROWS, G), jnp.int32),
            pltpu.VMEM((CHUNK_ROWS, G), jnp.int32),
            pltpu.VMEM((C, B_DIM), jnp.float32),
            pltpu.VMEM((C, FEAT), jnp.float32),
            pltpu.SemaphoreType.DMA,
            pltpu.SemaphoreType.DMA,
        ],
        compiler_params=pltpu.CompilerParams(use_tc_tiling_on_sc=False),
    )
    return kern(idx2d, feat2d, table, oe2d, oo2d)


def kernel(segmentId, pathSegmentFeat, table):
    idx2d = segmentId.astype(jnp.int32).reshape(TOTAL // G, G)
    feat2d = pathSegmentFeat.reshape(TOTAL, FEAT)
    oe2d = (2 * jnp.arange(TOTAL, dtype=jnp.int32)).reshape(TOTAL // G, G)
    oo2d = oe2d + 1
    out = _run(idx2d, feat2d, table, oe2d, oo2d)
    return out.reshape(N, L, B_DIM + FEAT)
